# baseline (speedup 1.0000x reference)
import functools

import jax
import jax.numpy as jnp
from jax import lax
from jax.experimental import pallas as pl
from jax.experimental.pallas import tpu as pltpu
from jax.experimental.pallas import tpu_sc as plsc

D = 64
EOI = 256000
SCALE = float(D) ** 0.5

NC, NS, LANES = 2, 16, 16
NW = NC * NS
CHUNK = 512


def _sc_embed(n_ids, table, ids, eoi):
    bpw = n_ids // NW
    nchunk = bpw // CHUNK
    mesh = plsc.VectorSubcoreMesh(core_axis_name="c", subcore_axis_name="s")

    @functools.partial(
        pl.kernel,
        out_type=jax.ShapeDtypeStruct((n_ids, D), jnp.float32),
        mesh=mesh,
        compiler_params=pltpu.CompilerParams(
            use_tc_tiling_on_sc=False, needs_layout_passes=False
        ),
        scratch_types=[
            pltpu.VMEM((bpw,), jnp.int32),
            pltpu.VMEM((CHUNK, D), jnp.float32),
            pltpu.VMEM((D,), jnp.float32),
            pltpu.SemaphoreType.DMA,
        ],
    )
    def body(table_hbm, ids_hbm, eoi_hbm, out_hbm, idx_v, rows, eoi_v, sem):
        wid = lax.axis_index("s") * NC + lax.axis_index("c")
        base = wid * bpw
        pltpu.sync_copy(ids_hbm.at[pl.ds(base, bpw)], idx_v)
        pltpu.sync_copy(eoi_hbm, eoi_v)
        eoi_regs = [eoi_v[pl.ds(j * LANES, LANES)] for j in range(D // LANES)]

        def chunk_body(c, carry):
            cbase = c * CHUNK
            pltpu.async_copy(
                table_hbm.at[idx_v.at[pl.ds(cbase, CHUNK)]], rows, sem
            ).wait()

            def group(g, gcarry):
                for r in range(LANES):
                    row = g * LANES + r
                    splat = plsc.load_gather(
                        idx_v,
                        [jnp.full((LANES,), cbase + row, jnp.int32)],
                    )
                    sel = splat == EOI
                    for j in range(D // LANES):
                        sl = pl.ds(j * LANES, LANES)
                        rows[row, sl] = jnp.where(sel, eoi_regs[j],
                                                  rows[row, sl] * SCALE)

                return gcarry

            lax.fori_loop(0, CHUNK // LANES, group, 0)
            pltpu.sync_copy(rows, out_hbm.at[pl.ds(base + cbase, CHUNK)])
            return carry

        lax.fori_loop(0, nchunk, chunk_body, 0)

    return body(table, ids, eoi)


def kernel(input_ids, embedding, eoi_embedding):
    ids = input_ids.reshape(-1)
    out = _sc_embed(ids.shape[0], embedding, ids, eoi_embedding)
    return out.reshape(input_ids.shape + (D,))


# speedup vs baseline: 1.2841x; 1.2841x over previous
import functools

import jax
import jax.numpy as jnp
from jax import lax
from jax.experimental import pallas as pl
from jax.experimental.pallas import tpu as pltpu
from jax.experimental.pallas import tpu_sc as plsc

D = 64
EOI = 256000
SCALE = float(D) ** 0.5

NC, NS, LANES = 2, 16, 16
NW = NC * NS
CHUNK = 512


def _splat(vec, idx):
    return lax.gather(
        vec,
        idx,
        lax.GatherDimensionNumbers(
            offset_dims=(), collapsed_slice_dims=(0,), start_index_map=(0,)
        ),
        (1,),
        mode=lax.GatherScatterMode.PROMISE_IN_BOUNDS,
    )


def _sc_embed(n_ids, table, ids, eoi):
    bpw = n_ids // NW
    nchunk = bpw // CHUNK
    mesh = plsc.VectorSubcoreMesh(core_axis_name="c", subcore_axis_name="s")

    @functools.partial(
        pl.kernel,
        out_type=jax.ShapeDtypeStruct((n_ids, D), jnp.float32),
        mesh=mesh,
        compiler_params=pltpu.CompilerParams(use_tc_tiling_on_sc=False),
        scratch_types=[
            pltpu.VMEM((bpw,), jnp.int32),
            pltpu.VMEM((CHUNK, D), jnp.float32),
            pltpu.VMEM((D,), jnp.float32),
            pltpu.SemaphoreType.DMA,
        ],
    )
    def body(table_hbm, ids_hbm, eoi_hbm, out_hbm, idx_v, rows, eoi_v, sem):
        wid = lax.axis_index("s") * NC + lax.axis_index("c")
        base = wid * bpw
        pltpu.sync_copy(ids_hbm.at[pl.ds(base, bpw)], idx_v)
        pltpu.sync_copy(eoi_hbm, eoi_v)
        eoi_regs = [eoi_v[pl.ds(j * LANES, LANES)] for j in range(D // LANES)]

        def chunk_body(c, carry):
            cbase = c * CHUNK
            pltpu.async_copy(
                table_hbm.at[idx_v.at[pl.ds(cbase, CHUNK)]], rows, sem
            ).wait()

            def group(g, gcarry):
                iv = idx_v[pl.ds(cbase + g * LANES, LANES)]
                bvec = jnp.where(iv == EOI, 1.0, 0.0).astype(jnp.float32)
                avec = SCALE - SCALE * bvec
                for r in range(LANES):
                    row = g * LANES + r
                    rsel = jnp.full((LANES, 1), r, jnp.int32)
                    a = _splat(avec, rsel)
                    b = _splat(bvec, rsel)
                    for j in range(D // LANES):
                        sl = pl.ds(j * LANES, LANES)
                        rows[row, sl] = rows[row, sl] * a + eoi_regs[j] * b

                return gcarry

            lax.fori_loop(0, CHUNK // LANES, group, 0)
            pltpu.sync_copy(rows, out_hbm.at[pl.ds(base + cbase, CHUNK)])
            return carry

        lax.fori_loop(0, nchunk, chunk_body, 0)

    return body(table, ids, eoi)


def kernel(input_ids, embedding, eoi_embedding):
    ids = input_ids.reshape(-1)
    out = _sc_embed(ids.shape[0], embedding, ids, eoi_embedding)
    return out.reshape(input_ids.shape + (D,))
